# baseline (speedup 1.0000x reference)
import functools

import jax
import jax.numpy as jnp
from jax import lax
from jax.experimental import pallas as pl
from jax.experimental.pallas import tpu as pltpu
from jax.experimental.pallas import tpu_sc as plsc

BATCH = 16384
D = 64
NC = 2
NS = 16
LANES = 16
NW = NC * NS
BPW = BATCH // NW


def _sc_body(uids_hbm, iids_hbm, uemb_hbm, iemb_hbm, z_hbm,
             uidx_v, iidx_v, urows_v, irows_v, sem_u, sem_i):
    wid = lax.axis_index("s") * NC + lax.axis_index("c")
    base = wid * BPW
    pltpu.sync_copy(uids_hbm.at[pl.ds(base, BPW)], uidx_v)
    pltpu.sync_copy(iids_hbm.at[pl.ds(base, BPW)], iidx_v)
    cu = pltpu.async_copy(uemb_hbm.at[uidx_v], urows_v, sem_u)
    ci = pltpu.async_copy(iemb_hbm.at[iidx_v], irows_v, sem_i)
    cu.wait()
    ci.wait()

    def row(r, carry):
        for c in range(D // LANES):
            sl = pl.ds(c * LANES, LANES)
            urows_v[r, sl] = urows_v[r, sl] * irows_v[r, sl]
        return carry

    lax.fori_loop(0, BPW, row, 0)
    pltpu.sync_copy(urows_v, z_hbm.at[pl.ds(base, BPW)])


@functools.partial(
    pl.kernel,
    mesh=plsc.VectorSubcoreMesh(core_axis_name="c", subcore_axis_name="s"),
    out_type=jax.ShapeDtypeStruct((BATCH, D), jnp.float32),
    scratch_types=[
        pltpu.VMEM((BPW,), jnp.int32),
        pltpu.VMEM((BPW,), jnp.int32),
        pltpu.VMEM((BPW, D), jnp.float32),
        pltpu.VMEM((BPW, D), jnp.float32),
        pltpu.SemaphoreType.DMA,
        pltpu.SemaphoreType.DMA,
    ],
    compiler_params=pltpu.CompilerParams(use_tc_tiling_on_sc=False),
)
def _sc_gather_mul(uids, iids, uemb, iemb, z, uidx_v, iidx_v, urows_v,
                   irows_v, sem_u, sem_i):
    _sc_body(uids, iids, uemb, iemb, z, uidx_v, iidx_v, urows_v, irows_v,
             sem_u, sem_i)


def _mlp_body(z_ref, w0_ref, b0_ref, w1_ref, b1_ref, hw_ref, hb_ref, out_ref):
    z = z_ref[...]
    h = lax.dot_general(z, w0_ref[...], (((1,), (1,)), ((), ())),
                        precision=lax.Precision.HIGHEST,
                        preferred_element_type=jnp.float32)
    h = jnp.maximum(h + b0_ref[...], 0.0)
    h = lax.dot_general(h, w1_ref[...], (((1,), (1,)), ((), ())),
                        precision=lax.Precision.HIGHEST,
                        preferred_element_type=jnp.float32)
    h = jnp.maximum(h + b1_ref[...], 0.0)
    out = jnp.sum(h * hw_ref[...], axis=1, keepdims=True)
    out_ref[...] = out + hb_ref[0, 0]


def _mlp(z, W0, b0, W1, b1, hW, hb):
    BLK = 2048
    return pl.pallas_call(
        _mlp_body,
        grid=(BATCH // BLK,),
        in_specs=[
            pl.BlockSpec((BLK, D), lambda i: (i, 0)),
            pl.BlockSpec((D, D), lambda i: (0, 0)),
            pl.BlockSpec((1, D), lambda i: (0, 0)),
            pl.BlockSpec((D, D), lambda i: (0, 0)),
            pl.BlockSpec((1, D), lambda i: (0, 0)),
            pl.BlockSpec((1, D), lambda i: (0, 0)),
            pl.BlockSpec(memory_space=pltpu.SMEM),
        ],
        out_specs=pl.BlockSpec((BLK, 1), lambda i: (i, 0)),
        out_shape=jax.ShapeDtypeStruct((BATCH, 1), jnp.float32),
    )(z, W0, b0.reshape(1, D), W1, b1.reshape(1, D), hW, hb.reshape(1, 1))


def kernel(user_ids, item_ids, user_emb, item_emb, W0, b0, W1, b1, hW, hb):
    uids = user_ids.astype(jnp.int32)
    iids = item_ids.astype(jnp.int32)
    z = _sc_gather_mul(uids, iids, user_emb, item_emb)
    return _mlp(z, W0, b0, W1, b1, hW, hb)


# speedup vs baseline: 1.6629x; 1.6629x over previous
import functools

import jax
import jax.numpy as jnp
from jax import lax
from jax.experimental import pallas as pl
from jax.experimental.pallas import tpu as pltpu
from jax.experimental.pallas import tpu_sc as plsc

BATCH = 16384
D = 64
NC = 2
NS = 16
LANES = 16
NW = NC * NS
BPW = BATCH // NW
PPW = BPW // 2
FIRE = 16


def _sc_body(uids_hbm, iids_hbm, uemb_hbm, iemb_hbm, z_hbm,
             uidx_v, iidx_v, urows_v, irows_v, sem_u, sem_i):
    wid = lax.axis_index("s") * NC + lax.axis_index("c")
    base = wid * BPW
    pltpu.sync_copy(uids_hbm.at[pl.ds(base, BPW)], uidx_v)
    pltpu.sync_copy(iids_hbm.at[pl.ds(base, BPW)], iidx_v)

    def fire(g, carry):
        r0 = g * FIRE
        uvec = uidx_v[pl.ds(r0, FIRE)]
        ivec = iidx_v[pl.ds(r0, FIRE)]
        for j in range(FIRE):
            dst_r = g * (FIRE // 2) + j // 2
            dst_c = pl.ds((j % 2) * D, D)
            pltpu.async_copy(uemb_hbm.at[uvec[j]], urows_v.at[dst_r, dst_c],
                             sem_u)
            pltpu.async_copy(iemb_hbm.at[ivec[j]], irows_v.at[dst_r, dst_c],
                             sem_i)
        return carry

    lax.fori_loop(0, BPW // FIRE, fire, 0)
    pltpu.make_async_copy(z_hbm.at[pl.ds(0, PPW)], urows_v, sem_u).wait()
    pltpu.make_async_copy(z_hbm.at[pl.ds(0, PPW)], irows_v, sem_i).wait()

    def row(r, carry):
        for c in range(128 // LANES):
            sl = pl.ds(c * LANES, LANES)
            urows_v[r, sl] = urows_v[r, sl] * irows_v[r, sl]
        return carry

    lax.fori_loop(0, PPW, row, 0)
    pltpu.sync_copy(urows_v, z_hbm.at[pl.ds(wid * PPW, PPW)])


@functools.partial(
    pl.kernel,
    mesh=plsc.VectorSubcoreMesh(core_axis_name="c", subcore_axis_name="s"),
    out_type=jax.ShapeDtypeStruct((BATCH // 2, 128), jnp.float32),
    scratch_types=[
        pltpu.VMEM((BPW,), jnp.int32),
        pltpu.VMEM((BPW,), jnp.int32),
        pltpu.VMEM((PPW, 128), jnp.float32),
        pltpu.VMEM((PPW, 128), jnp.float32),
        pltpu.SemaphoreType.DMA,
        pltpu.SemaphoreType.DMA,
    ],
)
def _sc_gather_mul(uids, iids, uemb, iemb, z, uidx_v, iidx_v, urows_v,
                   irows_v, sem_u, sem_i):
    _sc_body(uids, iids, uemb, iemb, z, uidx_v, iidx_v, urows_v, irows_v,
             sem_u, sem_i)


def _mlp_body(z_ref, w0_ref, b0_ref, w1_ref, b1_ref, hw_ref, hb_ref, out_ref):
    z = z_ref[...]
    h = lax.dot_general(z, w0_ref[...], (((1,), (0,)), ((), ())),
                        precision=lax.Precision.HIGHEST,
                        preferred_element_type=jnp.float32)
    h = jnp.maximum(h + b0_ref[...], 0.0)
    h = lax.dot_general(h, w1_ref[...], (((1,), (0,)), ((), ())),
                        precision=lax.Precision.HIGHEST,
                        preferred_element_type=jnp.float32)
    h = jnp.maximum(h + b1_ref[...], 0.0)
    e = h * hw_ref[...]
    s0 = jnp.sum(e[:, :D], axis=1, keepdims=True)
    s1 = jnp.sum(e[:, D:], axis=1, keepdims=True)
    out_ref[...] = jnp.concatenate([s0, s1], axis=1) + hb_ref[0, 0]


def _mlp(z, W0p, b0p, W1p, b1p, hWp, hb):
    BLK = 1024
    rows = BATCH // 2
    return pl.pallas_call(
        _mlp_body,
        grid=(rows // BLK,),
        in_specs=[
            pl.BlockSpec((BLK, 128), lambda i: (i, 0)),
            pl.BlockSpec((128, 128), lambda i: (0, 0)),
            pl.BlockSpec((1, 128), lambda i: (0, 0)),
            pl.BlockSpec((128, 128), lambda i: (0, 0)),
            pl.BlockSpec((1, 128), lambda i: (0, 0)),
            pl.BlockSpec((1, 128), lambda i: (0, 0)),
            pl.BlockSpec(memory_space=pltpu.SMEM),
        ],
        out_specs=pl.BlockSpec((BLK, 2), lambda i: (i, 0)),
        out_shape=jax.ShapeDtypeStruct((rows, 2), jnp.float32),
    )(z, W0p, b0p, W1p, b1p, hWp, hb.reshape(1, 1))


def kernel(user_ids, item_ids, user_emb, item_emb, W0, b0, W1, b1, hW, hb):
    uids = user_ids.astype(jnp.int32)
    iids = item_ids.astype(jnp.int32)
    z = _sc_gather_mul(uids, iids, user_emb, item_emb)

    zpad = jnp.zeros((D, D), jnp.float32)
    W0p = jnp.block([[W0.T, zpad], [zpad, W0.T]])
    W1p = jnp.block([[W1.T, zpad], [zpad, W1.T]])
    b0p = jnp.tile(b0.reshape(1, D), (1, 2))
    b1p = jnp.tile(b1.reshape(1, D), (1, 2))
    hWp = jnp.tile(hW.reshape(1, D), (1, 2))

    out2 = _mlp(z, W0p, b0p, W1p, b1p, hWp, hb)
    return out2.reshape(BATCH, 1)
